# baseline (speedup 1.0000x reference)
import jax
import jax.numpy as jnp
from jax.experimental import pallas as pl


def _final_matmul(pooled, Wlin, blin):
    G = pooled.shape[0]

    def body(p_ref, w_ref, b_ref, o_ref):
        o_ref[...] = p_ref[...] @ w_ref[...] + b_ref[0]

    out = pl.pallas_call(
        body,
        out_shape=jax.ShapeDtypeStruct((G, 1), jnp.float32),
    )(pooled, Wlin, blin)
    return out


def kernel(x, edge_index, batch, W1, b1, W2, b2, Wlin, blin):
    n = x.shape[0]
    G = 256
    loops = jnp.arange(n, dtype=edge_index.dtype)
    src = jnp.concatenate([edge_index[0], loops])
    dst = jnp.concatenate([edge_index[1], loops])
    deg = jax.ops.segment_sum(jnp.ones_like(src, dtype=x.dtype), dst, num_segments=n)
    dinv = jnp.where(deg > 0, 1.0 / jnp.sqrt(deg), 0.0)
    norm = dinv[src] * dinv[dst]

    def gcn_conv(h, W, b):
        h = h @ W
        msg = h[src] * norm[:, None]
        out = jax.ops.segment_sum(msg, dst, num_segments=n)
        return out + b

    h = jax.nn.relu(gcn_conv(x, W1, b1))
    h = jax.nn.relu(gcn_conv(h, W2, b2))
    counts = jax.ops.segment_sum(jnp.ones((n,), dtype=x.dtype), batch, num_segments=G)
    sums = jax.ops.segment_sum(h, batch, num_segments=G)
    pooled = sums / jnp.maximum(counts, 1.0)[:, None]
    out = _final_matmul(pooled, Wlin, blin)
    return out.squeeze()


# speedup vs baseline: 1.9197x; 1.9197x over previous
import functools

import jax
import jax.numpy as jnp
from jax import lax
from jax.experimental import pallas as pl
from jax.experimental.pallas import tpu as pltpu
from jax.experimental.pallas import tpu_sc as plsc

N = 100000
E = 1600000
F_IN = 16
H = 128
G = 256
NP = 100352
CHUNK = 8192
NCHUNK = 13
NT = 32
NS = 16
EPT = E // NT
IW = 80
ROWS_PER_TILE = EPT // IW
BLK_ROWS = 25
NBLK = ROWS_PER_TILE // BLK_ROWS
VPB = BLK_ROWS * IW // 16
CAP = E + NT * NCHUNK * 15 + 4096
ACC_ROWS = CHUNK + 16
DUMP = CHUNK + 8

_mesh = plsc.VectorSubcoreMesh(core_axis_name="c", subcore_axis_name="s")
_sc_params = pltpu.CompilerParams(needs_layout_passes=False,
                                  use_tc_tiling_on_sc=False)


def _zero16():
    return jnp.zeros((16,), jnp.float32)


@functools.partial(
    pl.kernel, mesh=_mesh,
    out_type=(jax.ShapeDtypeStruct((NT, 16), jnp.int32),
              jax.ShapeDtypeStruct((2 * NP,), jnp.float32)),
    scratch_types=[
        pltpu.VMEM((BLK_ROWS, IW), jnp.int32),
        pltpu.VMEM((IW,), jnp.float32),
        pltpu.VMEM((16,), jnp.int32),
        pltpu.VMEM((3136,), jnp.float32),
        pltpu.VMEM_SHARED((NP,), jnp.float32),
    ],
    compiler_params=_sc_params,
)
def _a1(dst2d_hbm, counts_hbm, deg_hbm, dstb, onesb, cbuf, zb, sh_deg):
    c = lax.axis_index("c")
    s = lax.axis_index("s")
    w = c * NS + s
    lanes = lax.iota(jnp.int32, 16)

    def _zb(i, _):
        zb[pl.ds(i * 16, 16)] = _zero16()
        return 0
    lax.fori_loop(0, 196, _zb, 0)
    for j in range(2):
        pltpu.sync_copy(zb, sh_deg.at[pl.ds(s * 6272 + j * 3136, 3136)])

    def _ones(i, _):
        onesb[pl.ds(i * 16, 16)] = jnp.ones((16,), jnp.float32)
        return 0
    lax.fori_loop(0, IW // 16, _ones, 0)
    plsc.subcore_barrier()

    def _block(blk, cnt):
        row0 = w * ROWS_PER_TILE + blk * BLK_ROWS
        pltpu.sync_copy(dst2d_hbm.at[pl.ds(row0, BLK_ROWS)], dstb)

        def _row(r, cnt):
            pltpu.sync_copy(onesb, sh_deg.at[dstb.at[r]], add=True)

            def _vreg(k, cnt):
                dv = dstb[r, pl.ds(k * 16, 16)]
                bv = lax.shift_right_logical(dv, 13)
                for bkt in range(NCHUNK):
                    m = bv == bkt
                    cpc = plsc.all_reduce_population_count(m)
                    cnt = jnp.where(lanes == bkt, cnt + cpc, cnt)
                return cnt
            return lax.fori_loop(0, IW // 16, _vreg, cnt)
        return lax.fori_loop(0, BLK_ROWS, _row, cnt)

    cnt = lax.fori_loop(0, NBLK, _block, jnp.zeros((16,), jnp.int32))
    cbuf[...] = cnt
    pltpu.sync_copy(cbuf, counts_hbm.at[w])
    plsc.subcore_barrier()
    for j in range(2):
        off = s * 6272 + j * 3136
        pltpu.sync_copy(sh_deg.at[pl.ds(off, 3136)],
                        deg_hbm.at[pl.ds(c * NP + off, 3136)])


def _a2_body(cnt_ref, off_ref):
    cnt = cnt_ref[...]
    pc = jnp.bitwise_and(cnt + 15, -16).astype(jnp.float32)
    tot = jnp.sum(pc, axis=0)
    bi = lax.broadcasted_iota(jnp.int32, (16, 16), 0)
    bj = lax.broadcasted_iota(jnp.int32, (16, 16), 1)
    mb = (bi < bj).astype(jnp.float32)
    base = jnp.dot(tot, mb, preferred_element_type=jnp.float32)
    ti = lax.broadcasted_iota(jnp.int32, (NT, NT), 0)
    tj = lax.broadcasted_iota(jnp.int32, (NT, NT), 1)
    mt = (tj < ti).astype(jnp.float32)
    rowoff = jnp.dot(mt, pc, preferred_element_type=jnp.float32)
    off_ref[...] = (base[None, :] + rowoff).astype(jnp.int32)


def _a2(counts):
    return pl.pallas_call(
        _a2_body,
        out_shape=jax.ShapeDtypeStruct((NT, 16), jnp.int32),
    )(counts)


@functools.partial(
    pl.kernel, mesh=_mesh,
    out_type=(jax.ShapeDtypeStruct((CAP,), jnp.int32),
              jax.ShapeDtypeStruct((CAP,), jnp.int32)),
    scratch_types=[
        pltpu.VMEM((BLK_ROWS, IW), jnp.int32),
        pltpu.VMEM((BLK_ROWS, IW), jnp.int32),
        pltpu.VMEM((BLK_ROWS, IW), jnp.int32),
        pltpu.VMEM((BLK_ROWS, IW), jnp.int32),
        pltpu.VMEM((BLK_ROWS, IW), jnp.int32),
        pltpu.VMEM((16,), jnp.int32),
    ],
    compiler_params=_sc_params,
)
def _a3(src2d_hbm, dst2d_hbm, off_hbm, srcp_hbm, dlp_hbm,
        srcb, dstb, stp, sts, std, cur_v):
    c = lax.axis_index("c")
    s = lax.axis_index("s")
    w = c * NS + s
    lanes = lax.iota(jnp.int32, 16)
    ones = jnp.ones((16,), jnp.int32)
    pltpu.sync_copy(off_hbm.at[w], cur_v)

    def _block(blk, _):
        row0 = w * ROWS_PER_TILE + blk * BLK_ROWS
        pltpu.sync_copy(src2d_hbm.at[pl.ds(row0, BLK_ROWS)], srcb)
        pltpu.sync_copy(dst2d_hbm.at[pl.ds(row0, BLK_ROWS)], dstb)

        def _vreg(v, _):
            r = v // (IW // 16)
            k = (v % (IW // 16)) * 16
            sv = srcb[r, pl.ds(k, 16)]
            dv = dstb[r, pl.ds(k, 16)]
            bv = lax.shift_right_logical(dv, 13)
            dl = jnp.bitwise_and(dv, CHUNK - 1)
            cur = cur_v[...]
            pos = jnp.zeros((16,), jnp.int32)
            upd = jnp.zeros((16,), jnp.int32)
            for bkt in range(NCHUNK):
                m = bv == bkt
                rank = plsc.cumsum(ones, mask=m)
                cpc = plsc.all_reduce_population_count(m)
                cur_b = jnp.broadcast_to(cur[bkt], (16,))
                pos = jnp.where(m, cur_b + rank - 1, pos)
                upd = jnp.where(lanes == bkt, upd + cpc, upd)
            cur_v[...] = cur + upd
            stp[r, pl.ds(k, 16)] = pos
            sts[r, pl.ds(k, 16)] = sv
            std[r, pl.ds(k, 16)] = dl
            return 0
        lax.fori_loop(0, VPB, _vreg, 0)

        def _flush(r, _):
            pltpu.sync_copy(sts.at[r], srcp_hbm.at[stp.at[r]])
            pltpu.sync_copy(std.at[r], dlp_hbm.at[stp.at[r]])
            return 0
        lax.fori_loop(0, BLK_ROWS, _flush, 0)
        return 0
    lax.fori_loop(0, NBLK, _block, 0)


def _make_layer(F):
    ZR = 171

    @functools.partial(
        pl.kernel, mesh=_mesh,
        out_type=jax.ShapeDtypeStruct((NP, F), jnp.float32),
        scratch_types=[
            pltpu.VMEM((NT, 16), jnp.int32),
            pltpu.VMEM((NT, 16), jnp.int32),
            pltpu.VMEM((ZR, F), jnp.float32),
            pltpu.VMEM((128,), jnp.int32),
            pltpu.VMEM((1, 128), jnp.int32),
            pltpu.VMEM((128, F), jnp.float32),
            pltpu.SemaphoreType.DMA,
            pltpu.VMEM_SHARED((ACC_ROWS, F), jnp.float32),
        ],
        compiler_params=_sc_params,
    )
    def _layer(g_hbm, srcp_hbm, dlp_hbm, off_hbm, cnt_hbm, s_hbm,
               off_v, cnt_v, zb, sidx, dlidx, rows, sem, sh_acc):
        c = lax.axis_index("c")
        s = lax.axis_index("s")
        lanes = lax.iota(jnp.int32, 16)
        pltpu.sync_copy(off_hbm, off_v)
        pltpu.sync_copy(cnt_hbm, cnt_v)

        def _zb(i, _):
            for j in range(F // 16):
                zb[i, pl.ds(j * 16, 16)] = _zero16()
            return 0
        lax.fori_loop(0, ZR, _zb, 0)

        def _run_chunk(bkt):
            for j in range(3):
                pltpu.sync_copy(zb, sh_acc.at[pl.ds(s * 513 + j * ZR, ZR)])
            plsc.subcore_barrier()

            def _seg(t, _):
                o_row = off_v[t, pl.ds(0, 16)]
                c_row = cnt_v[t, pl.ds(0, 16)]
                o = pl.multiple_of(o_row[bkt], 8)
                ln_all = c_row[bkt]
                q = pl.multiple_of(((ln_all + 127) // 128) * 8, 8)
                st = pl.multiple_of(s * q, 8)
                ln = jnp.clip(ln_all - st, 0, q)
                nblk = (ln + 127) // 128

                def _blk(b, _):
                    g0 = pl.multiple_of(o + st + b * 128, 8)
                    pltpu.sync_copy(srcp_hbm.at[pl.ds(g0, 128)], sidx)
                    pltpu.sync_copy(dlp_hbm.at[pl.ds(g0, 128)], dlidx.at[0])
                    rem = ln - b * 128

                    @pl.when(rem < 128)
                    def _():
                        for j in range(8):
                            m = (lanes + j * 16) < rem
                            sj = sidx[pl.ds(j * 16, 16)]
                            dj = dlidx[0, pl.ds(j * 16, 16)]
                            sidx[pl.ds(j * 16, 16)] = jnp.where(m, sj, 0)
                            dlidx[0, pl.ds(j * 16, 16)] = jnp.where(m, dj, DUMP)

                    pltpu.async_copy(g_hbm.at[sidx], rows, sem).wait()
                    pltpu.sync_copy(rows, sh_acc.at[dlidx.at[0]], add=True)
                    return 0
                lax.fori_loop(0, nblk, _blk, 0)
                return 0
            lax.fori_loop(0, NT, _seg, 0)
            plsc.subcore_barrier()

            if bkt != NCHUNK - 1:
                r0 = s * 512
                pltpu.sync_copy(sh_acc.at[pl.ds(r0, 512)],
                                s_hbm.at[pl.ds(bkt * CHUNK + r0, 512)])
            else:
                r0 = s * 128
                pltpu.sync_copy(sh_acc.at[pl.ds(r0, 128)],
                                s_hbm.at[pl.ds(bkt * CHUNK + r0, 128)])
            plsc.subcore_barrier()

        @pl.when(c == 0)
        def _():
            for bkt in range(0, NCHUNK, 2):
                _run_chunk(bkt)

        @pl.when(c == 1)
        def _():
            for bkt in range(1, NCHUNK, 2):
                _run_chunk(bkt)

    return _layer


_layer16 = _make_layer(16)
_layer128 = _make_layer(128)


BROWS = N // IW


@functools.partial(
    pl.kernel, mesh=_mesh,
    out_type=jax.ShapeDtypeStruct((G,), jnp.float32),
    scratch_types=[
        pltpu.VMEM((1, IW), jnp.int32),
        pltpu.VMEM((IW,), jnp.float32),
        pltpu.VMEM((IW,), jnp.float32),
        pltpu.VMEM((G,), jnp.float32),
        pltpu.VMEM((G,), jnp.float32),
        pltpu.VMEM((G,), jnp.float32),
        pltpu.VMEM_SHARED((G,), jnp.float32),
        pltpu.VMEM_SHARED((G,), jnp.float32),
    ],
    compiler_params=_sc_params,
)
def _pool(y_hbm, b2d_hbm, out_hbm, bidx, ybuf, onesb, sbuf, cbuf, obuf,
          sh_sum, sh_cnt):
    c = lax.axis_index("c")
    s = lax.axis_index("s")

    def _ones(i, _):
        onesb[pl.ds(i * 16, 16)] = jnp.ones((16,), jnp.float32)
        return 0
    lax.fori_loop(0, IW // 16, _ones, 0)

    @pl.when(jnp.logical_and(c == 0, s == 0))
    def _():
        def _zb(i, _):
            sbuf[pl.ds(i * 16, 16)] = _zero16()
            return 0
        lax.fori_loop(0, G // 16, _zb, 0)
        pltpu.sync_copy(sbuf, sh_sum)
        pltpu.sync_copy(sbuf, sh_cnt)
    plsc.subcore_barrier()

    @pl.when(c == 0)
    def _():
        r0 = s * 79
        nr = jnp.minimum(BROWS - r0, 79)

        def _row(i, _):
            r = r0 + i
            pltpu.sync_copy(b2d_hbm.at[r], bidx.at[0])
            pltpu.sync_copy(y_hbm.at[pl.ds(r * IW, IW)], ybuf)
            pltpu.sync_copy(ybuf, sh_sum.at[bidx.at[0]], add=True)
            pltpu.sync_copy(onesb, sh_cnt.at[bidx.at[0]], add=True)
            return 0
        lax.fori_loop(0, nr, _row, 0)
    plsc.subcore_barrier()

    @pl.when(jnp.logical_and(c == 0, s == 0))
    def _():
        pltpu.sync_copy(sh_sum, sbuf)
        pltpu.sync_copy(sh_cnt, cbuf)

        def _fin(i, _):
            sv = sbuf[pl.ds(i * 16, 16)]
            cv = cbuf[pl.ds(i * 16, 16)]
            obuf[pl.ds(i * 16, 16)] = sv / jnp.maximum(cv, 1.0)
            return 0
        lax.fori_loop(0, G // 16, _fin, 0)
        pltpu.sync_copy(obuf, out_hbm)


BS = 512
GRID = NP // BS


def _tc1_body(deg_ref, x_ref, dinv_ref, g1_ref):
    d = deg_ref[...]
    dt = d[0, :] + d[1, :] + 1.0
    dinv = lax.rsqrt(dt)
    dinv_ref[...] = dinv[:, None]
    g1_ref[...] = x_ref[...] * dinv[:, None]


def _tc1(deg2, x_p):
    return pl.pallas_call(
        _tc1_body,
        grid=(GRID,),
        in_specs=[
            pl.BlockSpec((2, BS), lambda i: (0, i)),
            pl.BlockSpec((BS, F_IN), lambda i: (i, 0)),
        ],
        out_specs=[
            pl.BlockSpec((BS, 1), lambda i: (i, 0)),
            pl.BlockSpec((BS, F_IN), lambda i: (i, 0)),
        ],
        out_shape=[
            jax.ShapeDtypeStruct((NP, 1), jnp.float32),
            jax.ShapeDtypeStruct((NP, F_IN), jnp.float32),
        ],
    )(deg2, x_p)


def _tc2_body(s1_ref, g1_ref, dinv_ref, w1_ref, b1_ref, w2_ref, g2_ref):
    dv = dinv_ref[...]
    pre = (s1_ref[...] + g1_ref[...]) * dv
    h1 = jnp.maximum(
        jnp.dot(pre, w1_ref[...], preferred_element_type=jnp.float32)
        + b1_ref[...], 0.0)
    g2_ref[...] = jnp.dot(h1, w2_ref[...],
                          preferred_element_type=jnp.float32) * dv


def _tc2(s1, g1, dinv, W1, b1, W2):
    return pl.pallas_call(
        _tc2_body,
        grid=(GRID,),
        in_specs=[
            pl.BlockSpec((BS, F_IN), lambda i: (i, 0)),
            pl.BlockSpec((BS, F_IN), lambda i: (i, 0)),
            pl.BlockSpec((BS, 1), lambda i: (i, 0)),
            pl.BlockSpec((F_IN, H), lambda i: (0, 0)),
            pl.BlockSpec((1, H), lambda i: (0, 0)),
            pl.BlockSpec((H, H), lambda i: (0, 0)),
        ],
        out_specs=pl.BlockSpec((BS, H), lambda i: (i, 0)),
        out_shape=jax.ShapeDtypeStruct((NP, H), jnp.float32),
    )(s1, g1, dinv, W1, b1.reshape(1, H), W2)


def _tc3_body(s2_ref, g2_ref, dinv_ref, b2_ref, wl_ref, y_ref):
    dv = dinv_ref[...]
    h2 = jnp.maximum((s2_ref[...] + g2_ref[...]) * dv + b2_ref[...], 0.0)
    y_ref[...] = jnp.dot(h2, wl_ref[...], preferred_element_type=jnp.float32)


def _tc3(s2, g2, dinv, b2, Wlin):
    return pl.pallas_call(
        _tc3_body,
        grid=(GRID,),
        in_specs=[
            pl.BlockSpec((BS, H), lambda i: (i, 0)),
            pl.BlockSpec((BS, H), lambda i: (i, 0)),
            pl.BlockSpec((BS, 1), lambda i: (i, 0)),
            pl.BlockSpec((1, H), lambda i: (0, 0)),
            pl.BlockSpec((H, 1), lambda i: (0, 0)),
        ],
        out_specs=pl.BlockSpec((BS, 1), lambda i: (i, 0)),
        out_shape=jax.ShapeDtypeStruct((NP, 1), jnp.float32),
    )(s2, g2, dinv, b2.reshape(1, H), Wlin)


def kernel(x, edge_index, batch, W1, b1, W2, b2, Wlin, blin):
    src2d = edge_index[0].reshape(E // IW, IW)
    dst2d = edge_index[1].reshape(E // IW, IW)

    counts, deg2 = _a1(dst2d)
    offsets = _a2(counts)
    srcp, dlp = _a3(src2d, dst2d, offsets)

    x_p = jnp.zeros((NP, F_IN), jnp.float32).at[:N].set(x)
    dinv, g1 = _tc1(deg2.reshape(2, NP), x_p)
    s1 = _layer16(g1, srcp, dlp, offsets, counts)
    g2 = _tc2(s1, g1, dinv, W1, b1, W2)
    s2 = _layer128(g2, srcp, dlp, offsets, counts)
    y = _tc3(s2, g2, dinv, b2, Wlin)
    pooled = _pool(y.reshape(NP), batch.reshape(BROWS, IW))
    return pooled + blin[0]
